# Initial kernel scaffold; baseline (speedup 1.0000x reference)
import jax
import jax.numpy as jnp
from jax.experimental import pallas as pl


def kernel(token_ids, context_features, special_table, cls_W, cls_b, cls_g, cls_beta, ctx_W, ctx_b, ctx_g, ctx_beta):
    raise NotImplementedError("write your pallas kernel here")


# speedup vs baseline: 4.3025x; 4.3025x over previous
import jax
import jax.numpy as jnp
from jax.experimental import pallas as pl

NUM_BET_BINS = 64
NUM_SPECIAL = 8
NUM_CONTEXT = 16
SPECIAL_OFFSET = NUM_BET_BINS
D_MODEL = 256
ROWS_PER_STEP = 2048


def _ln_relu(x, g, beta, eps=1e-5):
    mu = jnp.mean(x, axis=-1, keepdims=True)
    var = jnp.mean((x - mu) ** 2, axis=-1, keepdims=True)
    y = (x - mu) * jax.lax.rsqrt(var + eps) * g + beta
    return jnp.maximum(y, 0.0)


def _fused_kernel(tok_ref, cf_ref, table_ref, clsW_ref, clsb_ref, clsg_ref,
                  clsbeta_ref, ctxW_ref, ctxb_ref, ctxg_ref, ctxbeta_ref,
                  out_ref):
    tok = tok_ref[...]
    cf = cf_ref[...]

    ids = tok - SPECIAL_OFFSET
    special_mask = (ids >= 0) & (ids < NUM_SPECIAL)
    classes = jax.lax.broadcasted_iota(jnp.int32, (tok.shape[0], NUM_SPECIAL), 1)
    onehot = ((ids == classes) & special_mask).astype(jnp.float32)
    emb = jnp.dot(onehot, table_ref[...], preferred_element_type=jnp.float32)

    x_cls = jnp.dot(cf, clsW_ref[...], preferred_element_type=jnp.float32)
    x_cls = _ln_relu(x_cls + clsb_ref[...], clsg_ref[...], clsbeta_ref[...])
    cls_mask = (tok == SPECIAL_OFFSET + 0).astype(jnp.float32)
    emb = emb + cls_mask * x_cls

    x_ctx = jnp.dot(cf, ctxW_ref[...], preferred_element_type=jnp.float32)
    x_ctx = _ln_relu(x_ctx + ctxb_ref[...], ctxg_ref[...], ctxbeta_ref[...])
    ctx_mask = (tok == SPECIAL_OFFSET + 1).astype(jnp.float32)
    emb = emb + ctx_mask * x_ctx

    out_ref[...] = emb


@jax.jit
def kernel(token_ids, context_features, special_table, cls_W, cls_b, cls_g,
           cls_beta, ctx_W, ctx_b, ctx_g, ctx_beta):
    B, S = token_ids.shape
    n = B * S
    R = ROWS_PER_STEP
    grid = n // R

    tok2 = token_ids.reshape(n, 1)
    cf2 = context_features.reshape(n, NUM_CONTEXT)
    clsW_pad = jnp.zeros((NUM_CONTEXT, D_MODEL), cls_W.dtype).at[:3].set(cls_W)

    row_spec = lambda w: pl.BlockSpec((R, w), lambda i: (i, 0))
    full = lambda a: pl.BlockSpec(a.shape, lambda i: (0,) * a.ndim)
    vec = lambda v: v.reshape(1, -1)

    out = pl.pallas_call(
        _fused_kernel,
        grid=(grid,),
        in_specs=[
            row_spec(1),
            row_spec(NUM_CONTEXT),
            full(special_table),
            full(clsW_pad),
            pl.BlockSpec((1, D_MODEL), lambda i: (0, 0)),
            pl.BlockSpec((1, D_MODEL), lambda i: (0, 0)),
            pl.BlockSpec((1, D_MODEL), lambda i: (0, 0)),
            full(ctx_W),
            pl.BlockSpec((1, D_MODEL), lambda i: (0, 0)),
            pl.BlockSpec((1, D_MODEL), lambda i: (0, 0)),
            pl.BlockSpec((1, D_MODEL), lambda i: (0, 0)),
        ],
        out_specs=row_spec(D_MODEL),
        out_shape=jax.ShapeDtypeStruct((n, D_MODEL), jnp.float32),
    )(tok2, cf2, special_table, clsW_pad, vec(cls_b), vec(cls_g),
      vec(cls_beta), ctx_W, vec(ctx_b), vec(ctx_g), vec(ctx_beta))
    return out.reshape(B, S, D_MODEL)
